# baseline (speedup 1.0000x reference)
import functools
import math

import jax
import jax.numpy as jnp
from jax.experimental import pallas as pl
from jax.experimental.pallas import tpu as pltpu

_FORE = 12


def _dott(a, b):
    return jax.lax.dot_general(
        a, b, (((1,), (1,)), ((), ())), preferred_element_type=jnp.float32)


def _dgc_gru_kernel(
    feat_ref,
    pm_ref,
    adj_ref,
    ang_ref,
    wgcn_ref,
    wy_ref,
    wgi_ref,
    wee_ref,
    wgh_ref,
    fcw_ref,
    scal_ref,
    out_ref,
    hn_ref,
    xn_ref,
    c1_ref,
    c2_ref,
    *, n_nodes, nt, hid,
):
    i = pl.program_id(0)

    @pl.when(i == 0)
    def _init():
        ang = ang_ref[...]
        msk = (adj_ref[...] != 0).astype(jnp.float32)
        c1_ref[...] = jnp.cos(ang - (math.pi / 2.0)) * msk
        c2_ref[...] = jnp.cos(ang) * msk
        hn_ref[...] = jnp.concatenate(
            [jnp.zeros((hid, nt), jnp.float32),
             jnp.ones((1, nt), jnp.float32)], axis=0)
        xn_ref[...] = pm_ref[0]

    cf4 = feat_ref[...]
    cf_row = cf4.reshape(nt, cf4.shape[-1])
    xn_row = xn_ref[...]
    xn_flat = xn_row[0]
    hn_aug = hn_ref[...]
    hn_c = hn_aug[0:hid]

    w0x = scal_ref[0, 0]
    w1x = scal_ref[0, 1]
    conv_b = scal_ref[0, 2]
    fc_b = scal_ref[0, 3]

    u_col = cf4[0, 0, :, 0:1]
    v_col = cf4[0, 0, :, 1:2]
    w = ((v_col * c1_ref[...] + u_col * c2_ref[...]) >= 0.5
         ).astype(jnp.float32)
    deg = jnp.sum(w, axis=1)
    deg_safe = jnp.where(deg > 0, deg, 1.0)
    dis = jnp.where(deg > 0, 1.0 / jnp.sqrt(deg_safe), 0.0)
    y0 = (_dott(wy_ref[...], cf_row[0:n_nodes])[0]
          + xn_flat[0:n_nodes] * w1x)
    t = dis * y0
    g = -(dis * jnp.dot(t, w, preferred_element_type=jnp.float32))
    ge = jnp.concatenate([g, jnp.zeros((nt - n_nodes,), jnp.float32)])

    gcn = _dott(wgcn_ref[...], cf_row)[0] + xn_flat * w0x + ge + conv_b
    x_gcn = jax.nn.sigmoid(gcn)

    ee = jnp.concatenate(
        [xn_row, jnp.ones((1, nt), jnp.float32), x_gcn[None, :]],
        axis=0)
    gi = (_dott(wgi_ref[...], cf_row)
          + jnp.dot(wee_ref[...], ee, preferred_element_type=jnp.float32))
    gh = jnp.dot(wgh_ref[...], hn_aug,
                 preferred_element_type=jnp.float32)

    rz = jax.nn.sigmoid(gi[0:2 * hid] + gh[0:2 * hid])
    r = rz[0:hid]
    z = rz[hid:2 * hid]
    nn = jnp.tanh(gi[2 * hid:3 * hid] + r * gh[2 * hid:3 * hid])
    hn_new = nn + z * (hn_c - nn)

    xn_new = jnp.dot(fcw_ref[...], hn_new,
                     preferred_element_type=jnp.float32)[0] + fc_b

    hn_ref[0:hid, :] = hn_new
    xn_ref[...] = xn_new[None, :]
    out_ref[0, 0, :] = xn_new


def kernel(feature, pm25_hist, adj_mat, angles, conv_W0, conv_W1, conv_b,
           gru_Wih, gru_Whh, gru_bih, gru_bhh, fc_W, fc_b):
    B, T, N, D = feature.shape
    fore = _FORE
    hist = T - fore
    H = gru_Whh.shape[1]
    NT = B * N

    pm_last = pm25_hist[:, -1, :, 0].reshape(1, 1, NT)

    wgcn = conv_W0[1:].T
    wy = conv_W1[1:].T
    wgi = gru_Wih[:, 1:1 + D]
    wee = jnp.concatenate(
        [gru_Wih[:, 0:1], gru_bih[:, None], gru_Wih[:, 1 + D:2 + D]],
        axis=1)
    wgh = jnp.concatenate([gru_Whh, gru_bhh[:, None]], axis=1)
    fcw = fc_W.reshape(1, H)
    scal = jnp.stack([conv_W0[0, 0], conv_W1[0, 0],
                      conv_b[0], fc_b[0]]).reshape(1, 4)

    grid = (fore,)
    full = lambda shape: pl.BlockSpec(shape, lambda i: (0,) * len(shape))

    out = pl.pallas_call(
        functools.partial(_dgc_gru_kernel, n_nodes=N, nt=NT, hid=H),
        grid=grid,
        in_specs=[
            pl.BlockSpec((B, 1, N, D), lambda i: (0, hist + i, 0, 0)),
            pl.BlockSpec((1, 1, NT), lambda i: (0, 0, 0)),
            full((N, N)),
            full((N, N)),
            full((1, D)),
            full((1, D)),
            full((3 * H, D)),
            full((3 * H, 3)),
            full((3 * H, H + 1)),
            full((1, H)),
            full((1, 4)),
        ],
        out_specs=pl.BlockSpec((1, 1, NT), lambda i: (i, 0, 0)),
        out_shape=jax.ShapeDtypeStruct((fore, 1, NT), jnp.float32),
        scratch_shapes=[
            pltpu.VMEM((H + 1, NT), jnp.float32),
            pltpu.VMEM((1, NT), jnp.float32),
            pltpu.VMEM((N, N), jnp.float32),
            pltpu.VMEM((N, N), jnp.float32),
        ],
    )(feature, pm_last, adj_mat, angles, wgcn, wy, wgi, wee, wgh, fcw, scal)

    return out.reshape(fore, B, N).transpose(1, 0, 2)[..., None]


# speedup vs baseline: 2.4020x; 2.4020x over previous
import functools
import math

import jax
import jax.numpy as jnp
from jax.experimental import pallas as pl
from jax.experimental.pallas import tpu as pltpu

_FORE = 12


def _dgc_gru_kernel(
    feat_ref,
    wind_ref,
    pm_ref,
    adj_ref,
    ang_ref,
    wgcn_ref,
    wy_ref,
    wgi_ref,
    wgh_ref,
    fcw_ref,
    scal_ref,
    out_ref,
    hn_ref,
    xn_ref,
    c1_ref,
    c2_ref,
    *, n_nodes, nt, hid,
):
    i = pl.program_id(0)

    @pl.when(i == 0)
    def _init():
        ang = ang_ref[...]
        msk = (adj_ref[...] != 0).astype(jnp.float32)
        c1_ref[...] = jnp.cos(ang - (math.pi / 2.0)) * msk
        c2_ref[...] = jnp.cos(ang) * msk
        hn_ref[...] = jnp.concatenate(
            [jnp.zeros((hid, nt), jnp.float32),
             jnp.ones((1, nt), jnp.float32)], axis=0)
        xn_ref[...] = pm_ref[0]

    cf_t = feat_ref[0]
    xn_c = xn_ref[...]
    xn_flat = xn_c[0]
    hn_aug = hn_ref[...]
    hn_c = hn_aug[0:hid]
    fc_b = scal_ref[0, 0]
    w0x = scal_ref[0, 1]
    w1x = scal_ref[0, 2]
    conv_b = scal_ref[0, 3]

    def sigmoid(x):
        return 0.5 * jnp.tanh(0.5 * x) + 0.5

    u_col = wind_ref[0, :, 0:1]
    v_col = wind_ref[0, :, 1:2]
    w = ((v_col * c1_ref[...] + u_col * c2_ref[...]) >= 0.5
         ).astype(jnp.float32)
    deg = jnp.sum(w, axis=1)
    deg_safe = jnp.where(deg > 0, deg, 1.0)
    dis = jnp.where(deg > 0, 1.0 / jnp.sqrt(deg_safe), 0.0)
    y0 = (jnp.dot(wy_ref[...], cf_t[:, 0:n_nodes],
                  preferred_element_type=jnp.float32)[0]
          + xn_flat[0:n_nodes] * w1x)
    t = dis * y0
    g = -(dis * jnp.dot(t, w, preferred_element_type=jnp.float32))
    ge = jnp.concatenate([g, jnp.zeros((nt - n_nodes,), jnp.float32)])

    gcn = (jnp.dot(wgcn_ref[...], cf_t,
                   preferred_element_type=jnp.float32)[0]
           + xn_flat * w0x + ge + conv_b)
    x_gcn = sigmoid(gcn)

    aug = jnp.concatenate(
        [cf_t, xn_c, jnp.ones((1, nt), jnp.float32), x_gcn[None, :]],
        axis=0)
    gi = jnp.dot(wgi_ref[...], aug,
                 preferred_element_type=jnp.float32)
    gh = jnp.dot(wgh_ref[...], hn_aug,
                 preferred_element_type=jnp.float32)

    rz = sigmoid(gi[0:2 * hid] + gh[0:2 * hid])
    r = rz[0:hid]
    z = rz[hid:2 * hid]
    nn = jnp.tanh(gi[2 * hid:3 * hid] + r * gh[2 * hid:3 * hid])
    hn_new = nn + z * (hn_c - nn)

    xn_new = jnp.dot(fcw_ref[...], hn_new,
                     preferred_element_type=jnp.float32)[0] + fc_b

    hn_ref[0:hid, :] = hn_new
    xn_ref[...] = xn_new[None, :]
    out_ref[0, 0, :] = xn_new


def kernel(feature, pm25_hist, adj_mat, angles, conv_W0, conv_W1, conv_b,
           gru_Wih, gru_Whh, gru_bih, gru_bhh, fc_W, fc_b):
    B, T, N, D = feature.shape
    fore = _FORE
    hist = T - fore
    H = gru_Whh.shape[1]
    NT = B * N

    ftail = feature[:, hist:]
    feat = ftail.transpose(1, 3, 0, 2).reshape(fore, D, NT)
    wind = ftail[0, :, :, 0:2]
    pm_last = pm25_hist[:, -1, :, 0].reshape(1, 1, NT)

    wgcn = conv_W0[1:].T
    wy = conv_W1[1:].T
    wgi = jnp.concatenate(
        [gru_Wih[:, 1:1 + D], gru_Wih[:, 0:1], gru_bih[:, None],
         gru_Wih[:, 1 + D:2 + D]], axis=1)
    wgh = jnp.concatenate([gru_Whh, gru_bhh[:, None]], axis=1)
    fcw = fc_W.reshape(1, H)
    scal = jnp.stack([fc_b[0], conv_W0[0, 0], conv_W1[0, 0],
                      conv_b[0]]).reshape(1, 4)

    grid = (fore,)
    full = lambda shape: pl.BlockSpec(shape, lambda i: (0,) * len(shape))

    out = pl.pallas_call(
        functools.partial(_dgc_gru_kernel, n_nodes=N, nt=NT, hid=H),
        grid=grid,
        in_specs=[
            pl.BlockSpec((1, D, NT), lambda i: (i, 0, 0)),
            pl.BlockSpec((1, N, 2), lambda i: (i, 0, 0)),
            pl.BlockSpec((1, 1, NT), lambda i: (0, 0, 0)),
            full((N, N)),
            full((N, N)),
            full((1, D)),
            full((1, D)),
            full((3 * H, D + 3)),
            full((3 * H, H + 1)),
            full((1, H)),
            full((1, 4)),
        ],
        out_specs=pl.BlockSpec((1, 1, NT), lambda i: (i, 0, 0)),
        out_shape=jax.ShapeDtypeStruct((fore, 1, NT), jnp.float32),
        scratch_shapes=[
            pltpu.VMEM((H + 1, NT), jnp.float32),
            pltpu.VMEM((1, NT), jnp.float32),
            pltpu.VMEM((N, N), jnp.float32),
            pltpu.VMEM((N, N), jnp.float32),
        ],
    )(feat, wind, pm_last, adj_mat, angles, wgcn, wy, wgi, wgh, fcw, scal)

    return out.reshape(fore, B, N).transpose(1, 0, 2)[..., None]
